# Initial kernel scaffold; baseline (speedup 1.0000x reference)
import jax
import jax.numpy as jnp
from jax.experimental import pallas as pl


def kernel(user_id, timestamp_bucket, timestamp, customer_city, city_tokens, product_category, cat_tokens, user_table, ts_table, city_table, city_text_table, cat_table, cat_text_table, norm_mean, norm_var):
    raise NotImplementedError("write your pallas kernel here")


# speedup vs baseline: 1.5315x; 1.5315x over previous
import jax
import jax.numpy as jnp
from jax import lax
from jax.experimental import pallas as pl
from jax.experimental.pallas import tpu as pltpu
from jax.experimental.pallas import tpu_sc as plsc

D = 32
NC = 2
NS = 16
NW = NC * NS
CHUNK = 128
TOK = 4
OUT_W = 193


def _masked_avg_cols(tokv_ref, trows_ref, out_ref, r, off):
  tv = tokv_ref[pl.ds(4 * r, 16)]
  m = jnp.where(tv != 0, 1.0, 0.0).astype(jnp.float32)
  cnt = m[0] + m[1] + m[2] + m[3]
  inv = jnp.where(cnt < 1.5, 1.0,
                  jnp.where(cnt < 2.5, 0.5,
                            jnp.where(cnt < 3.5, jnp.float32(1.0 / 3.0), 0.25)))
  acc0 = jnp.zeros((16,), jnp.float32)
  acc1 = jnp.zeros((16,), jnp.float32)
  for t in range(TOK):
    w = m[t] * inv
    acc0 = acc0 + w * trows_ref[4 * r + t, pl.ds(0, 16)]
    acc1 = acc1 + w * trows_ref[4 * r + t, pl.ds(16, 16)]
  out_ref[r, pl.ds(off, 16)] = acc0
  out_ref[r, pl.ds(off + 16, 16)] = acc1


def _body(uid_hbm, tsb_hbm, ts_hbm, city_hbm, ctok_hbm, cat_hbm, gtok_hbm,
          user_t, ts_t, city_t, ctext_t, cat_t, gtext_t, mean_hbm, scale_hbm,
          out_hbm,
          idx_u, idx_ts, idx_city, idx_cat, ts_v, ctok_v, gtok_v,
          urows, tsrows, cityrows, catrows, ctrows, gtrows, out_v,
          mean_v, scale_v, sem):
  B = out_hbm.shape[0]
  rows_w = B // NW
  nchunk = rows_w // CHUNK

  wid = lax.axis_index("s") * NC + lax.axis_index("c")
  base = wid * rows_w

  pltpu.sync_copy(mean_hbm, mean_v)
  pltpu.sync_copy(scale_hbm, scale_v)
  mean_s = mean_v[...][0]
  scale_s = scale_v[...][0]

  def chunk_body(ci, carry):
    rbase = base + ci * CHUNK

    pltpu.sync_copy(uid_hbm.at[pl.ds(rbase, CHUNK)], idx_u)
    pltpu.sync_copy(tsb_hbm.at[pl.ds(rbase, CHUNK)], idx_ts)
    pltpu.sync_copy(city_hbm.at[pl.ds(rbase, CHUNK)], idx_city)
    pltpu.sync_copy(cat_hbm.at[pl.ds(rbase, CHUNK)], idx_cat)
    pltpu.sync_copy(ts_hbm.at[pl.ds(rbase, CHUNK)], ts_v.at[pl.ds(0, CHUNK)])
    pltpu.sync_copy(ctok_hbm.at[pl.ds(rbase * TOK, CHUNK * TOK)],
                    ctok_v.at[pl.ds(0, CHUNK * TOK)])
    pltpu.sync_copy(gtok_hbm.at[pl.ds(rbase * TOK, CHUNK * TOK)],
                    gtok_v.at[pl.ds(0, CHUNK * TOK)])

    cps = [
        pltpu.async_copy(user_t.at[idx_u], urows, sem),
        pltpu.async_copy(ts_t.at[idx_ts], tsrows, sem),
        pltpu.async_copy(city_t.at[idx_city], cityrows, sem),
        pltpu.async_copy(cat_t.at[idx_cat], catrows, sem),
    ]
    for k in range(TOK):
      cps.append(pltpu.async_copy(
          ctext_t.at[ctok_v.at[pl.ds(k * CHUNK, CHUNK)]],
          ctrows.at[pl.ds(k * CHUNK, CHUNK)], sem))
      cps.append(pltpu.async_copy(
          gtext_t.at[gtok_v.at[pl.ds(k * CHUNK, CHUNK)]],
          gtrows.at[pl.ds(k * CHUNK, CHUNK)], sem))
    for cp in cps:
      cp.wait()

    def row_body(r, carry2):
      out_v[r, pl.ds(0, 16)] = urows[r, pl.ds(0, 16)]
      out_v[r, pl.ds(16, 16)] = urows[r, pl.ds(16, 16)]
      out_v[r, pl.ds(32, 16)] = tsrows[r, pl.ds(0, 16)]
      out_v[r, pl.ds(48, 16)] = tsrows[r, pl.ds(16, 16)]
      tsv = ts_v[pl.ds(r, 16)]
      nt = (tsv[0] - mean_s) * scale_s
      out_v[r, pl.ds(64, 16)] = jnp.full((16,), nt, jnp.float32)
      out_v[r, pl.ds(65, 16)] = cityrows[r, pl.ds(0, 16)]
      out_v[r, pl.ds(81, 16)] = cityrows[r, pl.ds(16, 16)]
      _masked_avg_cols(ctok_v, ctrows, out_v, r, 97)
      out_v[r, pl.ds(129, 16)] = catrows[r, pl.ds(0, 16)]
      out_v[r, pl.ds(145, 16)] = catrows[r, pl.ds(16, 16)]
      _masked_avg_cols(gtok_v, gtrows, out_v, r, 161)
      return carry2

    lax.fori_loop(0, CHUNK, row_body, 0)

    pltpu.sync_copy(out_v, out_hbm.at[pl.ds(rbase, CHUNK)])
    return carry

  lax.fori_loop(0, nchunk, chunk_body, 0)


def kernel(user_id, timestamp_bucket, timestamp, customer_city, city_tokens,
           product_category, cat_tokens, user_table, ts_table, city_table,
           city_text_table, cat_table, cat_text_table, norm_mean, norm_var):
  B = user_id.shape[0]
  scale = jax.lax.rsqrt(norm_var + 1e-7)
  mean16 = jnp.full((16,), norm_mean, jnp.float32)
  scale16 = jnp.full((16,), scale, jnp.float32)

  run = pl.kernel(
      _body,
      out_type=jax.ShapeDtypeStruct((B, OUT_W), jnp.float32),
      mesh=plsc.VectorSubcoreMesh(core_axis_name="c", subcore_axis_name="s"),
      compiler_params=pltpu.CompilerParams(use_tc_tiling_on_sc=False),
      scratch_types=[
          pltpu.VMEM((CHUNK,), jnp.int32),
          pltpu.VMEM((CHUNK,), jnp.int32),
          pltpu.VMEM((CHUNK,), jnp.int32),
          pltpu.VMEM((CHUNK,), jnp.int32),
          pltpu.VMEM((CHUNK + 16,), jnp.float32),
          pltpu.VMEM((CHUNK * TOK + 16,), jnp.int32),
          pltpu.VMEM((CHUNK * TOK + 16,), jnp.int32),
          pltpu.VMEM((CHUNK, D), jnp.float32),
          pltpu.VMEM((CHUNK, D), jnp.float32),
          pltpu.VMEM((CHUNK, D), jnp.float32),
          pltpu.VMEM((CHUNK, D), jnp.float32),
          pltpu.VMEM((CHUNK * TOK, D), jnp.float32),
          pltpu.VMEM((CHUNK * TOK, D), jnp.float32),
          pltpu.VMEM((CHUNK, OUT_W), jnp.float32),
          pltpu.VMEM((16,), jnp.float32),
          pltpu.VMEM((16,), jnp.float32),
          pltpu.SemaphoreType.DMA,
      ],
  )
  return run(user_id, timestamp_bucket, timestamp, customer_city,
             city_tokens.reshape(-1), product_category, cat_tokens.reshape(-1),
             user_table, ts_table, city_table, city_text_table, cat_table,
             cat_text_table, mean16, scale16)


# speedup vs baseline: 1.9863x; 1.2970x over previous
import jax
import jax.numpy as jnp
from jax import lax
from jax.experimental import pallas as pl
from jax.experimental.pallas import tpu as pltpu
from jax.experimental.pallas import tpu_sc as plsc

D = 32
NC = 2
NS = 16
NW = NC * NS
CHUNK = 128
TOK = 4
OUT_W = 193


def _masked_avg_cols(tokv_ref, trows_ref, out_ref, r, off):
  tv = tokv_ref[pl.ds(4 * r, 16)]
  m = jnp.where(tv != 0, 1.0, 0.0).astype(jnp.float32)
  cnt = m[0] + m[1] + m[2] + m[3]
  inv = jnp.where(cnt < 1.5, 1.0,
                  jnp.where(cnt < 2.5, 0.5,
                            jnp.where(cnt < 3.5, jnp.float32(1.0 / 3.0), 0.25)))
  acc0 = jnp.zeros((16,), jnp.float32)
  acc1 = jnp.zeros((16,), jnp.float32)
  for t in range(TOK):
    w = m[t] * inv
    acc0 = acc0 + w * trows_ref[4 * r + t, pl.ds(0, 16)]
    acc1 = acc1 + w * trows_ref[4 * r + t, pl.ds(16, 16)]
  out_ref[r, pl.ds(off, 16)] = acc0
  out_ref[r, pl.ds(off + 16, 16)] = acc1


NBUF = 16


def _user_body(uid_hbm, user_t, ue_hbm, idx_u, tiles, outbuf, sem):
  B = ue_hbm.shape[0]
  rows_w = B // NW
  nchunk = rows_w // CHUNK

  wid = lax.axis_index("s") * NC + lax.axis_index("c")
  base = wid * rows_w

  def chunk_body(ci, carry):
    rbase = base + ci * CHUNK
    pltpu.sync_copy(uid_hbm.at[pl.ds(rbase, CHUNK)],
                    idx_u.at[pl.ds(0, CHUNK)])

    def group_body(g, carry2):
      j0 = g * NBUF
      rs = []
      cps = []
      for b in range(NBUF):
        r = idx_u[pl.ds(j0 + b, 16)][0]
        rs.append(r)
        base8 = pl.multiple_of((r >> 3) << 3, 8)
        cps.append(pltpu.async_copy(user_t.at[pl.ds(base8, 8)],
                                    tiles.at[b], sem))
      for b in range(NBUF):
        cps[b].wait()
        rm8 = rs[b] & 7
        outbuf[j0 + b, pl.ds(0, 16)] = tiles[b, rm8, pl.ds(0, 16)]
        outbuf[j0 + b, pl.ds(16, 16)] = tiles[b, rm8, pl.ds(16, 16)]
      return carry2

    lax.fori_loop(0, CHUNK // NBUF, group_body, 0)
    pltpu.sync_copy(outbuf, ue_hbm.at[pl.ds(rbase, CHUNK)])
    return carry

  lax.fori_loop(0, nchunk, chunk_body, 0)


def _body(ue_in_hbm, tsb_hbm, ts_hbm, city_hbm, ctok_hbm, cat_hbm, gtok_hbm,
          ts_t, city_t, ctext_t, cat_t, gtext_t, mean_hbm, scale_hbm,
          out_hbm,
          idx_ts, idx_city, idx_cat, ts_v, ctok_v, gtok_v,
          urows, tsrows, cityrows, catrows, ctrows, gtrows, out_v,
          mean_v, scale_v, sem):
  B = out_hbm.shape[0]
  rows_w = B // NW
  nchunk = rows_w // CHUNK

  wid = lax.axis_index("s") * NC + lax.axis_index("c")
  base = wid * rows_w

  pltpu.sync_copy(mean_hbm, mean_v)
  pltpu.sync_copy(scale_hbm, scale_v)
  mean_s = mean_v[...][0]
  scale_s = scale_v[...][0]

  def chunk_body(ci, carry):
    rbase = base + ci * CHUNK

    pltpu.sync_copy(ue_in_hbm.at[pl.ds(rbase, CHUNK)], urows)
    pltpu.sync_copy(tsb_hbm.at[pl.ds(rbase, CHUNK)], idx_ts)
    pltpu.sync_copy(city_hbm.at[pl.ds(rbase, CHUNK)], idx_city)
    pltpu.sync_copy(cat_hbm.at[pl.ds(rbase, CHUNK)], idx_cat)
    pltpu.sync_copy(ts_hbm.at[pl.ds(rbase, CHUNK)], ts_v.at[pl.ds(0, CHUNK)])
    pltpu.sync_copy(ctok_hbm.at[pl.ds(rbase * TOK, CHUNK * TOK)],
                    ctok_v.at[pl.ds(0, CHUNK * TOK)])
    pltpu.sync_copy(gtok_hbm.at[pl.ds(rbase * TOK, CHUNK * TOK)],
                    gtok_v.at[pl.ds(0, CHUNK * TOK)])

    cps = [
        pltpu.async_copy(ts_t.at[idx_ts], tsrows, sem),
        pltpu.async_copy(city_t.at[idx_city], cityrows, sem),
        pltpu.async_copy(cat_t.at[idx_cat], catrows, sem),
    ]
    for k in range(TOK):
      cps.append(pltpu.async_copy(
          ctext_t.at[ctok_v.at[pl.ds(k * CHUNK, CHUNK)]],
          ctrows.at[pl.ds(k * CHUNK, CHUNK)], sem))
      cps.append(pltpu.async_copy(
          gtext_t.at[gtok_v.at[pl.ds(k * CHUNK, CHUNK)]],
          gtrows.at[pl.ds(k * CHUNK, CHUNK)], sem))
    for cp in cps:
      cp.wait()

    def row_body(r, carry2):
      out_v[r, pl.ds(0, 16)] = urows[r, pl.ds(0, 16)]
      out_v[r, pl.ds(16, 16)] = urows[r, pl.ds(16, 16)]
      out_v[r, pl.ds(32, 16)] = tsrows[r, pl.ds(0, 16)]
      out_v[r, pl.ds(48, 16)] = tsrows[r, pl.ds(16, 16)]
      tsv = ts_v[pl.ds(r, 16)]
      nt = (tsv[0] - mean_s) * scale_s
      out_v[r, pl.ds(64, 16)] = jnp.full((16,), nt, jnp.float32)
      out_v[r, pl.ds(65, 16)] = cityrows[r, pl.ds(0, 16)]
      out_v[r, pl.ds(81, 16)] = cityrows[r, pl.ds(16, 16)]
      _masked_avg_cols(ctok_v, ctrows, out_v, r, 97)
      out_v[r, pl.ds(129, 16)] = catrows[r, pl.ds(0, 16)]
      out_v[r, pl.ds(145, 16)] = catrows[r, pl.ds(16, 16)]
      _masked_avg_cols(gtok_v, gtrows, out_v, r, 161)
      return carry2

    lax.fori_loop(0, CHUNK, row_body, 0)

    pltpu.sync_copy(out_v, out_hbm.at[pl.ds(rbase, CHUNK)])
    return carry

  lax.fori_loop(0, nchunk, chunk_body, 0)


def kernel(user_id, timestamp_bucket, timestamp, customer_city, city_tokens,
           product_category, cat_tokens, user_table, ts_table, city_table,
           city_text_table, cat_table, cat_text_table, norm_mean, norm_var):
  B = user_id.shape[0]
  scale = jax.lax.rsqrt(norm_var + 1e-7)
  mean16 = jnp.full((16,), norm_mean, jnp.float32)
  scale16 = jnp.full((16,), scale, jnp.float32)

  gather_user = pl.kernel(
      _user_body,
      out_type=jax.ShapeDtypeStruct((B, D), jnp.float32),
      mesh=plsc.VectorSubcoreMesh(core_axis_name="c", subcore_axis_name="s"),
      compiler_params=pltpu.CompilerParams(use_tc_tiling_on_sc=True),
      scratch_types=[
          pltpu.VMEM((CHUNK + 16,), jnp.int32),
          pltpu.VMEM((NBUF, 8, D), jnp.float32),
          pltpu.VMEM((CHUNK, D), jnp.float32),
          pltpu.SemaphoreType.DMA,
      ],
  )
  ue = gather_user(user_id, user_table)

  run = pl.kernel(
      _body,
      out_type=jax.ShapeDtypeStruct((B, OUT_W), jnp.float32),
      mesh=plsc.VectorSubcoreMesh(core_axis_name="c", subcore_axis_name="s"),
      compiler_params=pltpu.CompilerParams(use_tc_tiling_on_sc=False),
      scratch_types=[
          pltpu.VMEM((CHUNK,), jnp.int32),
          pltpu.VMEM((CHUNK,), jnp.int32),
          pltpu.VMEM((CHUNK,), jnp.int32),
          pltpu.VMEM((CHUNK + 16,), jnp.float32),
          pltpu.VMEM((CHUNK * TOK + 16,), jnp.int32),
          pltpu.VMEM((CHUNK * TOK + 16,), jnp.int32),
          pltpu.VMEM((CHUNK, D), jnp.float32),
          pltpu.VMEM((CHUNK, D), jnp.float32),
          pltpu.VMEM((CHUNK, D), jnp.float32),
          pltpu.VMEM((CHUNK, D), jnp.float32),
          pltpu.VMEM((CHUNK * TOK, D), jnp.float32),
          pltpu.VMEM((CHUNK * TOK, D), jnp.float32),
          pltpu.VMEM((CHUNK, OUT_W), jnp.float32),
          pltpu.VMEM((16,), jnp.float32),
          pltpu.VMEM((16,), jnp.float32),
          pltpu.SemaphoreType.DMA,
      ],
  )
  return run(ue, timestamp_bucket, timestamp, customer_city,
             city_tokens.reshape(-1), product_category, cat_tokens.reshape(-1),
             ts_table, city_table, city_text_table, cat_table,
             cat_text_table, mean16, scale16)
